# baseline (speedup 1.0000x reference)
import functools

import jax
import jax.numpy as jnp
from jax import lax
from jax.experimental import pallas as pl
from jax.experimental.pallas import tpu as pltpu
from jax.experimental.pallas import tpu_sc as plsc

_BATCH = 16384
_EMB = 64
_BACKEND = 128
_NCLS = 128
_VOCAB = 100000

_HALF = 8192
_UNROLL = 4


def _build_gather_t():
    info = plsc.get_sparse_core_info()
    nw = info.num_cores * info.num_subcores
    rounds = _EMB // nw
    n_half = _BATCH // _HALF
    groups = _HALF // 16
    mesh = plsc.VectorSubcoreMesh(core_axis_name="c", subcore_axis_name="s")

    @functools.partial(
        pl.kernel,
        mesh=mesh,
        compiler_params=pltpu.CompilerParams(
            needs_layout_passes=False,
            disable_bounds_checks=True,
            disable_semaphore_checks=True),
        out_type=jax.ShapeDtypeStruct((_EMB, _BATCH), jnp.float32),
        scratch_types=[
            pltpu.VMEM((_VOCAB,), jnp.float32),
            pltpu.VMEM((_BATCH,), jnp.int32),
            pltpu.VMEM((_HALF,), jnp.float32),
        ],
    )
    def gather_t(idx_hbm, table_t_hbm, out_hbm, row_v, idx_v, out_v):
        wid = lax.axis_index("s") * info.num_cores + lax.axis_index("c")
        pltpu.sync_copy(idx_hbm, idx_v)

        def row_body(r, _):
            f = wid * rounds + r
            pltpu.sync_copy(table_t_hbm.at[f], row_v)

            def half_body(h, _):
                @plsc.parallel_loop(0, groups * 16, step=16 * _UNROLL)
                def _(base):
                    for u in range(_UNROLL):
                        out_v[pl.ds(base + u * 16, 16)] = (
                            plsc.load_gather(
                                row_v,
                                [idx_v[pl.ds(h * _HALF + base + u * 16, 16)]])
                        )
                pltpu.sync_copy(out_v, out_hbm.at[f, pl.ds(h * _HALF, _HALF)])
                return 0

            lax.fori_loop(0, n_half, half_body, 0)
            return 0

        lax.fori_loop(0, rounds, row_body, 0)

    return gather_t


_gather_t = _build_gather_t()


def _mm_body(x_ref, ft_ref, w1_ref, w2_ref, b_ref, o_ref):
    o_ref[...] = (
        jnp.dot(x_ref[...], w1_ref[...], preferred_element_type=jnp.float32)
        + lax.dot_general(
            ft_ref[...], w2_ref[...], (((0,), (0,)), ((), ())),
            preferred_element_type=jnp.float32)
        + b_ref[...]
    )


def kernel(batched_data, class_emb, W, b, domains):
    table_t = class_emb.T
    feat_t = _gather_t(domains, table_t)

    w1t = W[:, :_BACKEND].T
    w2t = W[:, _BACKEND:].T
    b2d = b.reshape(1, _NCLS)

    bm = 4096
    out = pl.pallas_call(
        _mm_body,
        grid=(_BATCH // bm,),
        in_specs=[
            pl.BlockSpec((bm, _BACKEND), lambda i: (i, 0)),
            pl.BlockSpec((_EMB, bm), lambda i: (0, i)),
            pl.BlockSpec((_BACKEND, _NCLS), lambda i: (0, 0)),
            pl.BlockSpec((_EMB, _NCLS), lambda i: (0, 0)),
            pl.BlockSpec((1, _NCLS), lambda i: (0, 0)),
        ],
        out_specs=pl.BlockSpec((bm, _NCLS), lambda i: (i, 0)),
        out_shape=jax.ShapeDtypeStruct((_BATCH, _NCLS), jnp.float32),
    )(batched_data, feat_t, w1t, w2t, b2d)
    return out


# speedup vs baseline: 1.0377x; 1.0377x over previous
import functools

import jax
import jax.numpy as jnp
from jax import lax
from jax.experimental import pallas as pl
from jax.experimental.pallas import tpu as pltpu
from jax.experimental.pallas import tpu_sc as plsc

_BATCH = 16384
_EMB = 64
_BACKEND = 128
_NCLS = 128
_VOCAB = 100000

_HALF = 8192
_UNROLL = 4


def _build_gather_t():
    info = plsc.get_sparse_core_info()
    nw = info.num_cores * info.num_subcores
    rounds = _EMB // nw
    n_half = _BATCH // _HALF
    groups = _HALF // 16
    mesh = plsc.VectorSubcoreMesh(core_axis_name="c", subcore_axis_name="s")

    @functools.partial(
        pl.kernel,
        mesh=mesh,
        compiler_params=pltpu.CompilerParams(
            needs_layout_passes=False,
            disable_bounds_checks=True,
            disable_semaphore_checks=True),
        out_type=jax.ShapeDtypeStruct((_EMB, _BATCH), jnp.float32),
        scratch_types=[
            pltpu.VMEM((_VOCAB,), jnp.float32),
            pltpu.VMEM((_BATCH,), jnp.int32),
            pltpu.VMEM((_HALF,), jnp.float32),
        ],
    )
    def gather_t(idx_hbm, table_t_hbm, out_hbm, row_v, idx_v, out_v):
        wid = lax.axis_index("s") * info.num_cores + lax.axis_index("c")
        pltpu.sync_copy(idx_hbm, idx_v)

        def row_body(r, _):
            f = wid * rounds + r
            pltpu.sync_copy(table_t_hbm.at[f], row_v)

            def half_body(h, _):
                @plsc.parallel_loop(0, groups * 16, step=16 * _UNROLL)
                def _(base):
                    for u in range(_UNROLL):
                        out_v[pl.ds(base + u * 16, 16)] = (
                            plsc.load_gather(
                                row_v,
                                [idx_v[pl.ds(h * _HALF + base + u * 16, 16)]])
                        )
                pltpu.sync_copy(out_v, out_hbm.at[f, pl.ds(h * _HALF, _HALF)])
                return 0

            lax.fori_loop(0, n_half, half_body, 0)
            return 0

        lax.fori_loop(0, rounds, row_body, 0)

    return gather_t


_gather_t = _build_gather_t()


def _mm_body(x_ref, ft_ref, w1_ref, w2_ref, b_ref, o_ref):
    xb = x_ref[...].astype(jnp.bfloat16)
    fb = ft_ref[...].astype(jnp.bfloat16)
    o_ref[...] = (
        jnp.dot(xb, w1_ref[...].astype(jnp.bfloat16),
                preferred_element_type=jnp.float32)
        + lax.dot_general(
            fb, w2_ref[...].astype(jnp.bfloat16), (((0,), (0,)), ((), ())),
            preferred_element_type=jnp.float32)
        + b_ref[...]
    )


def kernel(batched_data, class_emb, W, b, domains):
    table_t = class_emb.T
    feat_t = _gather_t(domains, table_t)

    w1t = W[:, :_BACKEND].T
    w2t = W[:, _BACKEND:].T
    b2d = b.reshape(1, _NCLS)

    bm = 8192
    out = pl.pallas_call(
        _mm_body,
        grid=(_BATCH // bm,),
        in_specs=[
            pl.BlockSpec((bm, _BACKEND), lambda i: (i, 0)),
            pl.BlockSpec((_EMB, bm), lambda i: (0, i)),
            pl.BlockSpec((_BACKEND, _NCLS), lambda i: (0, 0)),
            pl.BlockSpec((_EMB, _NCLS), lambda i: (0, 0)),
            pl.BlockSpec((1, _NCLS), lambda i: (0, 0)),
        ],
        out_specs=pl.BlockSpec((bm, _NCLS), lambda i: (i, 0)),
        out_shape=jax.ShapeDtypeStruct((_BATCH, _NCLS), jnp.float32),
    )(batched_data, feat_t, w1t, w2t, b2d)
    return out
